# baseline (speedup 1.0000x reference)
import functools

import jax
import jax.numpy as jnp
from jax import lax
from jax.experimental import pallas as pl
from jax.experimental.pallas import tpu as pltpu
from jax.experimental.pallas import tpu_sc as plsc

_N = 10000
_E = 16000
_HID = 32
_NCORE = 2
_NSUB = 16
_NW = _NCORE * _NSUB
_EPAD = 16384
_EPT = _EPAD // _NSUB
_CHK = 128
_NCHK = _EPT // _CHK
_HALF = _N // _NCORE
_SROWS = 5008
_ZROW = 5000
_TRASH = 5001
_ZCH = _SROWS // _NSUB
_ETRASH = _EPAD - 4
_UCH = 125
_NUCH = _HALF // _UCH
_DIMS = ((32, 32), (32, 32), (32, 64), (64, 64), (64, 64))

_mesh = plsc.VectorSubcoreMesh(core_axis_name="c", subcore_axis_name="s")
_sc_params = pltpu.CompilerParams(use_tc_tiling_on_sc=False)


@functools.cache
def _gather0_fn(ci):
    @functools.partial(
        pl.kernel,
        out_type=jax.ShapeDtypeStruct((_EPAD, ci), jnp.float32),
        mesh=_mesh,
        scratch_types=[
            pltpu.VMEM((_NCHK // 2, _CHK), jnp.int32),
            pltpu.VMEM((_EPAD // _NW, ci), jnp.float32),
            pltpu.SemaphoreType.DMA,
        ],
        compiler_params=_sc_params,
    )
    def gat(table, idx, out, idx_v, rows_v, sem):
        epw = _EPAD // _NW
        w = lax.axis_index("s") * _NCORE + lax.axis_index("c")
        pltpu.sync_copy(idx.at[w], idx_v)
        descs = [
            pltpu.async_copy(
                table.at[idx_v.at[j]], rows_v.at[pl.ds(j * _CHK, _CHK)], sem
            )
            for j in range(_NCHK // 2)
        ]
        for d in descs:
            d.wait()
        pltpu.sync_copy(rows_v, out.at[pl.ds(w * epw, epw)])

    return gat


def _clamped_idx(raw_v, idx_v, base, miss_row):
    for k in range(_EPT // 16):
        v = raw_v[pl.ds(k * 16, 16)]
        owned = (v >= base) & (v < base + _HALF)
        loc = jnp.where(owned, v - base, miss_row)
        idx_v[k // 8, pl.ds((k % 8) * 16, 16)] = loc


@functools.cache
def _sc_fused_fn(co, relu, gather):
    out_type = [jax.ShapeDtypeStruct((_N, co), jnp.float32)]
    if gather:
        out_type.append(jax.ShapeDtypeStruct((_EPAD, co), jnp.float32))

    @functools.partial(
        pl.kernel,
        out_type=tuple(out_type),
        mesh=_mesh,
        scratch_types=[
            pltpu.VMEM((_EPT,), jnp.int32),
            pltpu.VMEM((_NCHK, _CHK), jnp.int32),
            pltpu.VMEM((_NCHK, _CHK), jnp.int32),
            pltpu.VMEM((_EPT, co), jnp.float32),
            pltpu.VMEM((_UCH, co), jnp.float32),
            pltpu.VMEM((_UCH, co), jnp.float32),
            pltpu.VMEM_SHARED((_SROWS, co), jnp.float32),
            pltpu.SemaphoreType.DMA,
        ],
        compiler_params=_sc_params,
    )
    def fused(msg, dst, src, r_in, zeros, *rest):
        if gather:
            (h_out, xj_out, raw_v, idx_v, pos_v, dat_v, rbuf, abuf, spbuf,
             sem) = rest
        else:
            h_out, raw_v, idx_v, pos_v, dat_v, rbuf, abuf, spbuf, sem = rest
        c = lax.axis_index("c")
        s = lax.axis_index("s")
        base = c * _HALF

        d_raw = pltpu.async_copy(dst.at[pl.ds(s * _EPT, _EPT)], raw_v, sem)
        d_msg = pltpu.async_copy(msg.at[pl.ds(s * _EPT, _EPT)], dat_v, sem)
        d_z = pltpu.async_copy(
            zeros.at[pl.ds(s * _ZCH, _ZCH)], spbuf.at[pl.ds(s * _ZCH, _ZCH)],
            sem,
        )
        d_raw.wait()
        _clamped_idx(raw_v, idx_v, base, _TRASH)
        d_msg.wait()
        d_z.wait()
        plsc.subcore_barrier()

        descs = [
            pltpu.async_copy(
                dat_v.at[pl.ds(j * _CHK, _CHK)],
                spbuf.at[idx_v.at[j]],
                sem,
                add=True,
            )
            for j in range(_NCHK)
        ]
        for d in descs:
            d.wait()
        plsc.subcore_barrier()

        for k in range(_NUCH // _NSUB + 1):
            cid = s + _NSUB * k

            @pl.when(cid < _NUCH)
            def _():
                row0 = cid * _UCH
                pltpu.sync_copy(r_in.at[pl.ds(base + row0, _UCH)], rbuf)
                pltpu.sync_copy(spbuf.at[pl.ds(row0, _UCH)], abuf)

                def upd(i, carry):
                    for l in range(co // 16):
                        vv = (
                            abuf[i, pl.ds(l * 16, 16)]
                            + rbuf[i, pl.ds(l * 16, 16)]
                        )
                        if relu:
                            vv = jnp.maximum(vv, 0.0)
                        abuf[i, pl.ds(l * 16, 16)] = vv
                    return carry

                lax.fori_loop(0, _UCH, upd, 0)
                pltpu.sync_copy(abuf, h_out.at[pl.ds(base + row0, _UCH)])
                pltpu.sync_copy(abuf, spbuf.at[pl.ds(row0, _UCH)])

        if gather:
            plsc.subcore_barrier()
            pltpu.sync_copy(src.at[pl.ds(s * _EPT, _EPT)], raw_v)
            lanes = lax.iota(jnp.int32, 16)
            for k in range(_EPT // 16):
                v = raw_v[pl.ds(k * 16, 16)]
                owned = (v >= base) & (v < base + _HALF)
                idx_v[k // 8, pl.ds((k % 8) * 16, 16)] = jnp.where(
                    owned, v - base, _ZROW
                )
                pos = s * _EPT + k * 16 + lanes
                pos_v[k // 8, pl.ds((k % 8) * 16, 16)] = jnp.where(
                    owned, pos, _ETRASH
                )
            descs2 = [
                pltpu.async_copy(
                    spbuf.at[idx_v.at[j]],
                    dat_v.at[pl.ds(j * _CHK, _CHK)],
                    sem,
                )
                for j in range(_NCHK)
            ]
            for d in descs2:
                d.wait()
            descs3 = [
                pltpu.async_copy(
                    dat_v.at[pl.ds(j * _CHK, _CHK)],
                    xj_out.at[pos_v.at[j]],
                    sem,
                )
                for j in range(_NCHK)
            ]
            for d in descs3:
                d.wait()

    return fused


@functools.cache
def _msgr_fn(ci, co, nparts):
    be = 1024
    grid = _EPAD // be
    bn = 640

    def body(*refs):
        (*xs, eat_ref, w1_ref, b1_ref, w2t_ref, b2_ref,
         h_ref, root_ref, bias_ref, o_ref, r_ref) = refs
        i = pl.program_id(0)
        rows = lax.broadcasted_iota(jnp.int32, (be, 1), 0) + i * be
        valid = rows < _E
        xj = xs[0][...]
        for xr in xs[1:]:
            xj = xj + xr[...]
        xj = jnp.where(valid, xj, 0.0)
        validt = (
            lax.broadcasted_iota(jnp.int32, (1, be), 1) + i * be
        ) < _E
        hidt = jnp.maximum(w1_ref[...] * eat_ref[...] + b1_ref[...], 0.0)
        hidt = jnp.where(validt, hidt, 0.0)
        tp = lax.dot_general(
            w2t_ref[...], xj.astype(jnp.bfloat16), (((1,), (1,)), ((), ())),
            preferred_element_type=jnp.float32,
        )
        msgt = hidt[0:1, :] * tp[0:co, :]
        for h in range(1, _HID):
            msgt = msgt + hidt[h:h + 1, :] * tp[h * co:(h + 1) * co, :]
        msg = jnp.transpose(msgt, (1, 0)) + lax.dot_general(
            xj, b2_ref[...], (((1,), (0,)), ((), ())),
            preferred_element_type=jnp.float32,
        )
        o_ref[...] = msg
        r_ref[...] = (
            lax.dot_general(
                h_ref[...], root_ref[...], (((1,), (0,)), ((), ())),
                preferred_element_type=jnp.float32,
                precision=lax.Precision.HIGHEST,
            )
            + bias_ref[...]
        )

    xspecs = [pl.BlockSpec((be, ci), lambda i: (i, 0))] * nparts
    return pl.pallas_call(
        body,
        grid=(grid,),
        in_specs=xspecs + [
            pl.BlockSpec((1, be), lambda i: (0, i)),
            pl.BlockSpec((_HID, 1), lambda i: (0, 0)),
            pl.BlockSpec((_HID, 1), lambda i: (0, 0)),
            pl.BlockSpec((_HID * co, ci), lambda i: (0, 0)),
            pl.BlockSpec((ci, co), lambda i: (0, 0)),
            pl.BlockSpec((bn, ci), lambda i: (i, 0)),
            pl.BlockSpec((ci, co), lambda i: (0, 0)),
            pl.BlockSpec((1, co), lambda i: (0, 0)),
        ],
        out_specs=[
            pl.BlockSpec((be, co), lambda i: (i, 0)),
            pl.BlockSpec((bn, co), lambda i: (i, 0)),
        ],
        out_shape=[
            jax.ShapeDtypeStruct((_EPAD, co), jnp.float32),
            jax.ShapeDtypeStruct((_N, co), jnp.float32),
        ],
    )


@functools.cache
def _pool_fn(emb, nc):
    def body(h_ref, fcw_ref, fcb_ref, o_ref):
        m = jnp.max(h_ref[...], axis=0, keepdims=True)
        o_ref[...] = (
            lax.dot_general(
                jnp.maximum(m, 0.0), fcw_ref[...], (((1,), (0,)), ((), ())),
                preferred_element_type=jnp.float32,
                precision=lax.Precision.HIGHEST,
            )
            + fcb_ref[...]
        )

    return pl.pallas_call(
        body,
        out_shape=jax.ShapeDtypeStruct((1, nc), jnp.float32),
    )


def kernel(x, edge_attr, params, edge_index, batch):
    pad = _EPAD - _E
    src = jnp.concatenate([edge_index[0], jnp.zeros((pad,), jnp.int32)])
    dst = jnp.concatenate([edge_index[1], jnp.zeros((pad,), jnp.int32)])
    src3d = src.reshape(_NW, _NCHK // 2, _CHK)
    eap = jnp.concatenate([edge_attr, jnp.zeros((pad, 1), jnp.float32)])
    zeros64 = jnp.zeros((_SROWS, 64), jnp.float32)

    h = x
    xparts = [_gather0_fn(_DIMS[0][0])(x, src3d)]
    out = None
    eat = eap.reshape(1, _EPAD)
    for i, (ci, co) in enumerate(_DIMS):
        p = params["l%d" % i]
        w2t = (
            p["w2"].reshape(_HID, ci, co).transpose(0, 2, 1)
            .reshape(_HID * co, ci).astype(jnp.bfloat16)
        )
        msg, r = _msgr_fn(ci, co, len(xparts))(
            *xparts, eat, p["w1"].reshape(_HID, 1), p["b1"].reshape(_HID, 1),
            w2t, p["b2"].reshape(ci, co), h, p["root"],
            p["bias"].reshape(1, co),
        )
        last = i + 1 == len(_DIMS)
        res = _sc_fused_fn(co, not last, not last)(
            msg, dst, src, r, zeros64[:, :co]
        )
        if last:
            h = res[0]
        else:
            h, xj = res
            xparts = [xj]
    nc = params["fc_w"].shape[1]
    return _pool_fn(64, nc)(h, params["fc_w"], params["fc_b"].reshape(1, nc))


# speedup vs baseline: 2.7401x; 2.7401x over previous
import functools

import jax
import jax.numpy as jnp
from jax import lax
from jax.experimental import pallas as pl
from jax.experimental.pallas import tpu as pltpu
from jax.experimental.pallas import tpu_sc as plsc

_N = 10000
_E = 16000
_HID = 32
_NCORE = 2
_NSUB = 16
_NW = _NCORE * _NSUB
_EPAD = 16384
_EPT = _EPAD // _NSUB
_CHK = 128
_NCHK = _EPT // _CHK
_HALF = _N // _NCORE
_SROWS = 5008
_ZROW = 5000
_TRASH = 5001
_ZCH = _SROWS // _NSUB
_UCH = 125
_NUCH = _HALF // _UCH
_DIMS = ((32, 32), (32, 32), (32, 64), (64, 64), (64, 64))

_mesh = plsc.VectorSubcoreMesh(core_axis_name="c", subcore_axis_name="s")
_sc_params = pltpu.CompilerParams(use_tc_tiling_on_sc=False)


@functools.cache
def _gather0_fn(ci):
    @functools.partial(
        pl.kernel,
        out_type=jax.ShapeDtypeStruct((_EPAD, ci), jnp.float32),
        mesh=_mesh,
        scratch_types=[
            pltpu.VMEM((_NCHK // 2, _CHK), jnp.int32),
            pltpu.VMEM((_EPAD // _NW, ci), jnp.float32),
            pltpu.SemaphoreType.DMA,
        ],
        compiler_params=_sc_params,
    )
    def gat(table, idx, out, idx_v, rows_v, sem):
        epw = _EPAD // _NW
        w = lax.axis_index("s") * _NCORE + lax.axis_index("c")
        pltpu.sync_copy(idx.at[w], idx_v)
        descs = [
            pltpu.async_copy(
                table.at[idx_v.at[j]], rows_v.at[pl.ds(j * _CHK, _CHK)], sem
            )
            for j in range(_NCHK // 2)
        ]
        for d in descs:
            d.wait()
        pltpu.sync_copy(rows_v, out.at[pl.ds(w * epw, epw)])

    return gat


def _clamped_idx(raw_v, idx_v, base, miss_row):
    for k in range(_EPT // 16):
        v = raw_v[pl.ds(k * 16, 16)]
        owned = (v >= base) & (v < base + _HALF)
        loc = jnp.where(owned, v - base, miss_row)
        idx_v[k // 8, pl.ds((k % 8) * 16, 16)] = loc


@functools.cache
def _sc_fused_fn(co, relu, gather):
    out_type = [jax.ShapeDtypeStruct((_N, co), jnp.float32)]
    if gather:
        out_type.append(jax.ShapeDtypeStruct((_NCORE, _EPAD, co), jnp.float32))

    @functools.partial(
        pl.kernel,
        out_type=tuple(out_type),
        mesh=_mesh,
        scratch_types=[
            pltpu.VMEM((_EPT,), jnp.int32),
            pltpu.VMEM((_NCHK, _CHK), jnp.int32),
            pltpu.VMEM((_EPT, co), jnp.float32),
            pltpu.VMEM((_UCH, co), jnp.float32),
            pltpu.VMEM((_UCH, co), jnp.float32),
            pltpu.VMEM_SHARED((_SROWS, co), jnp.float32),
            pltpu.SemaphoreType.DMA,
        ],
        compiler_params=_sc_params,
    )
    def fused(msg, dst, src, r_in, zeros, *rest):
        if gather:
            h_out, xj_out, raw_v, idx_v, dat_v, rbuf, abuf, spbuf, sem = rest
        else:
            h_out, raw_v, idx_v, dat_v, rbuf, abuf, spbuf, sem = rest
        c = lax.axis_index("c")
        s = lax.axis_index("s")
        base = c * _HALF

        d_raw = pltpu.async_copy(dst.at[pl.ds(s * _EPT, _EPT)], raw_v, sem)
        d_msg = pltpu.async_copy(msg.at[pl.ds(s * _EPT, _EPT)], dat_v, sem)
        d_z = pltpu.async_copy(
            zeros.at[pl.ds(s * _ZCH, _ZCH)], spbuf.at[pl.ds(s * _ZCH, _ZCH)],
            sem,
        )
        d_raw.wait()
        _clamped_idx(raw_v, idx_v, base, _TRASH)
        d_msg.wait()
        d_z.wait()
        plsc.subcore_barrier()

        descs = [
            pltpu.async_copy(
                dat_v.at[pl.ds(j * _CHK, _CHK)],
                spbuf.at[idx_v.at[j]],
                sem,
                add=True,
            )
            for j in range(_NCHK)
        ]
        for d in descs:
            d.wait()
        plsc.subcore_barrier()

        for k in range(_NUCH // _NSUB + 1):
            cid = s + _NSUB * k

            @pl.when(cid < _NUCH)
            def _():
                row0 = cid * _UCH
                pltpu.sync_copy(r_in.at[pl.ds(base + row0, _UCH)], rbuf)
                pltpu.sync_copy(spbuf.at[pl.ds(row0, _UCH)], abuf)

                def upd(i, carry):
                    for l in range(co // 16):
                        vv = (
                            abuf[i, pl.ds(l * 16, 16)]
                            + rbuf[i, pl.ds(l * 16, 16)]
                        )
                        if relu:
                            vv = jnp.maximum(vv, 0.0)
                        abuf[i, pl.ds(l * 16, 16)] = vv
                    return carry

                lax.fori_loop(0, _UCH, upd, 0)
                pltpu.sync_copy(abuf, h_out.at[pl.ds(base + row0, _UCH)])
                pltpu.sync_copy(abuf, spbuf.at[pl.ds(row0, _UCH)])

        if gather:
            plsc.subcore_barrier()
            pltpu.sync_copy(src.at[pl.ds(s * _EPT, _EPT)], raw_v)
            _clamped_idx(raw_v, idx_v, base, _ZROW)
            descs2 = [
                pltpu.async_copy(
                    spbuf.at[idx_v.at[j]],
                    dat_v.at[pl.ds(j * _CHK, _CHK)],
                    sem,
                )
                for j in range(_NCHK)
            ]
            for d in descs2:
                d.wait()
            pltpu.sync_copy(dat_v, xj_out.at[c].at[pl.ds(s * _EPT, _EPT)])

    return fused


@functools.cache
def _msgr_fn(ci, co, nparts):
    be = 1024
    grid = _EPAD // be
    bn = 640

    def body(*refs):
        (*xs, eat_ref, w1_ref, b1_ref, w2t_ref, b2_ref,
         h_ref, root_ref, bias_ref, o_ref, r_ref) = refs
        i = pl.program_id(0)
        rows = lax.broadcasted_iota(jnp.int32, (be, 1), 0) + i * be
        valid = rows < _E
        xj = xs[0][...]
        for xr in xs[1:]:
            xj = xj + xr[...]
        xj = jnp.where(valid, xj, 0.0)
        validt = (
            lax.broadcasted_iota(jnp.int32, (1, be), 1) + i * be
        ) < _E
        hidt = jnp.maximum(w1_ref[...] * eat_ref[...] + b1_ref[...], 0.0)
        hidt = jnp.where(validt, hidt, 0.0)
        tp = lax.dot_general(
            w2t_ref[...], xj.astype(jnp.bfloat16), (((1,), (1,)), ((), ())),
            preferred_element_type=jnp.float32,
        )
        msgt = hidt[0:1, :] * tp[0:co, :]
        for h in range(1, _HID):
            msgt = msgt + hidt[h:h + 1, :] * tp[h * co:(h + 1) * co, :]
        msg = jnp.transpose(msgt, (1, 0)) + lax.dot_general(
            xj, b2_ref[...], (((1,), (0,)), ((), ())),
            preferred_element_type=jnp.float32,
        )
        o_ref[...] = msg
        r_ref[...] = (
            lax.dot_general(
                h_ref[...], root_ref[...], (((1,), (0,)), ((), ())),
                preferred_element_type=jnp.float32,
                precision=lax.Precision.HIGHEST,
            )
            + bias_ref[...]
        )

    xspecs = [pl.BlockSpec((be, ci), lambda i: (i, 0))] * nparts
    return pl.pallas_call(
        body,
        grid=(grid,),
        in_specs=xspecs + [
            pl.BlockSpec((1, be), lambda i: (0, i)),
            pl.BlockSpec((_HID, 1), lambda i: (0, 0)),
            pl.BlockSpec((_HID, 1), lambda i: (0, 0)),
            pl.BlockSpec((_HID * co, ci), lambda i: (0, 0)),
            pl.BlockSpec((ci, co), lambda i: (0, 0)),
            pl.BlockSpec((bn, ci), lambda i: (i, 0)),
            pl.BlockSpec((ci, co), lambda i: (0, 0)),
            pl.BlockSpec((1, co), lambda i: (0, 0)),
        ],
        out_specs=[
            pl.BlockSpec((be, co), lambda i: (i, 0)),
            pl.BlockSpec((bn, co), lambda i: (i, 0)),
        ],
        out_shape=[
            jax.ShapeDtypeStruct((_EPAD, co), jnp.float32),
            jax.ShapeDtypeStruct((_N, co), jnp.float32),
        ],
    )


@functools.cache
def _pool_fn(emb, nc):
    def body(h_ref, fcw_ref, fcb_ref, o_ref):
        m = jnp.max(h_ref[...], axis=0, keepdims=True)
        o_ref[...] = (
            lax.dot_general(
                jnp.maximum(m, 0.0), fcw_ref[...], (((1,), (0,)), ((), ())),
                preferred_element_type=jnp.float32,
                precision=lax.Precision.HIGHEST,
            )
            + fcb_ref[...]
        )

    return pl.pallas_call(
        body,
        out_shape=jax.ShapeDtypeStruct((1, nc), jnp.float32),
    )


def kernel(x, edge_attr, params, edge_index, batch):
    pad = _EPAD - _E
    src = jnp.concatenate([edge_index[0], jnp.zeros((pad,), jnp.int32)])
    dst = jnp.concatenate([edge_index[1], jnp.zeros((pad,), jnp.int32)])
    src3d = src.reshape(_NW, _NCHK // 2, _CHK)
    eap = jnp.concatenate([edge_attr, jnp.zeros((pad, 1), jnp.float32)])
    zeros64 = jnp.zeros((_SROWS, 64), jnp.float32)

    h = x
    xparts = [_gather0_fn(_DIMS[0][0])(x, src3d)]
    out = None
    eat = eap.reshape(1, _EPAD)
    for i, (ci, co) in enumerate(_DIMS):
        p = params["l%d" % i]
        w2t = (
            p["w2"].reshape(_HID, ci, co).transpose(0, 2, 1)
            .reshape(_HID * co, ci).astype(jnp.bfloat16)
        )
        msg, r = _msgr_fn(ci, co, len(xparts))(
            *xparts, eat, p["w1"].reshape(_HID, 1), p["b1"].reshape(_HID, 1),
            w2t, p["b2"].reshape(ci, co), h, p["root"],
            p["bias"].reshape(1, co),
        )
        last = i + 1 == len(_DIMS)
        res = _sc_fused_fn(co, not last, not last)(
            msg, dst, src, r, zeros64[:, :co]
        )
        if last:
            h = res[0]
        else:
            h, xj2 = res
            xparts = [xj2[0], xj2[1]]
    nc = params["fc_w"].shape[1]
    return _pool_fn(64, nc)(h, params["fc_w"], params["fc_b"].reshape(1, nc))


# speedup vs baseline: 2.9945x; 1.0928x over previous
import functools

import jax
import jax.numpy as jnp
from jax import lax
from jax.experimental import pallas as pl
from jax.experimental.pallas import tpu as pltpu
from jax.experimental.pallas import tpu_sc as plsc

_N = 10000
_E = 16000
_HID = 32
_NCORE = 2
_NSUB = 16
_NW = _NCORE * _NSUB
_EPAD = 16384
_EPT = _EPAD // _NSUB
_CHK = 128
_NCHK = _EPT // _CHK
_HALF = _N // _NCORE
_SROWS = 5008
_ZROW = 5000
_TRASH = 5001
_ZCH = _SROWS // _NSUB
_UCH = 125
_NUCH = _HALF // _UCH
_DIMS = ((32, 32), (32, 32), (32, 64), (64, 64), (64, 64))

_mesh = plsc.VectorSubcoreMesh(core_axis_name="c", subcore_axis_name="s")
_sc_params = pltpu.CompilerParams(use_tc_tiling_on_sc=False)


@functools.cache
def _gather0_fn(ci):
    @functools.partial(
        pl.kernel,
        out_type=jax.ShapeDtypeStruct((_EPAD, ci), jnp.float32),
        mesh=_mesh,
        scratch_types=[
            pltpu.VMEM((_NCHK // 2, _CHK), jnp.int32),
            pltpu.VMEM((_EPAD // _NW, ci), jnp.float32),
            pltpu.SemaphoreType.DMA,
        ],
        compiler_params=_sc_params,
    )
    def gat(table, idx, out, idx_v, rows_v, sem):
        epw = _EPAD // _NW
        w = lax.axis_index("s") * _NCORE + lax.axis_index("c")
        pltpu.sync_copy(idx.at[w], idx_v)
        descs = [
            pltpu.async_copy(
                table.at[idx_v.at[j]], rows_v.at[pl.ds(j * _CHK, _CHK)], sem
            )
            for j in range(_NCHK // 2)
        ]
        for d in descs:
            d.wait()
        pltpu.sync_copy(rows_v, out.at[pl.ds(w * epw, epw)])

    return gat


def _clamped_idx(raw_v, idx_v, base, miss_row):
    for k in range(_EPT // 16):
        v = raw_v[pl.ds(k * 16, 16)]
        owned = (v >= base) & (v < base + _HALF)
        loc = jnp.where(owned, v - base, miss_row)
        idx_v[k // 8, pl.ds((k % 8) * 16, 16)] = loc


@functools.cache
def _sc_fused_fn(co, relu, gather):
    out_type = [jax.ShapeDtypeStruct((_N, co), jnp.float32)]
    if gather:
        out_type.append(jax.ShapeDtypeStruct((_NCORE, _EPAD, co), jnp.float32))

    @functools.partial(
        pl.kernel,
        out_type=tuple(out_type),
        mesh=_mesh,
        scratch_types=[
            pltpu.VMEM((_EPT,), jnp.int32),
            pltpu.VMEM((_NCHK, _CHK), jnp.int32),
            pltpu.VMEM((_EPT, co), jnp.float32),
            pltpu.VMEM((_UCH, co), jnp.float32),
            pltpu.VMEM((_UCH, co), jnp.float32),
            pltpu.VMEM_SHARED((_SROWS, co), jnp.float32),
            pltpu.SemaphoreType.DMA,
        ],
        compiler_params=_sc_params,
    )
    def fused(msg, dst, src, r_in, zeros, *rest):
        if gather:
            h_out, xj_out, raw_v, idx_v, dat_v, rbuf, abuf, spbuf, sem = rest
        else:
            h_out, raw_v, idx_v, dat_v, rbuf, abuf, spbuf, sem = rest
        c = lax.axis_index("c")
        s = lax.axis_index("s")
        base = c * _HALF

        d_raw = pltpu.async_copy(dst.at[pl.ds(s * _EPT, _EPT)], raw_v, sem)
        d_msg = pltpu.async_copy(msg.at[pl.ds(s * _EPT, _EPT)], dat_v, sem)
        d_z = pltpu.async_copy(
            zeros.at[pl.ds(s * _ZCH, _ZCH)], spbuf.at[pl.ds(s * _ZCH, _ZCH)],
            sem,
        )
        d_raw.wait()
        _clamped_idx(raw_v, idx_v, base, _TRASH)
        d_msg.wait()
        d_z.wait()
        plsc.subcore_barrier()

        descs = [
            pltpu.async_copy(
                dat_v.at[pl.ds(j * _CHK, _CHK)],
                spbuf.at[idx_v.at[j]],
                sem,
                add=True,
            )
            for j in range(_NCHK)
        ]
        for d in descs:
            d.wait()
        plsc.subcore_barrier()

        for k in range(_NUCH // _NSUB + 1):
            cid = s + _NSUB * k

            @pl.when(cid < _NUCH)
            def _():
                row0 = cid * _UCH
                pltpu.sync_copy(r_in.at[pl.ds(base + row0, _UCH)], rbuf)
                pltpu.sync_copy(spbuf.at[pl.ds(row0, _UCH)], abuf)

                def upd(i, carry):
                    for l in range(co // 16):
                        vv = (
                            abuf[i, pl.ds(l * 16, 16)]
                            + rbuf[i, pl.ds(l * 16, 16)]
                        )
                        if relu:
                            vv = jnp.maximum(vv, 0.0)
                        abuf[i, pl.ds(l * 16, 16)] = vv
                    return carry

                lax.fori_loop(0, _UCH, upd, 0)
                pltpu.sync_copy(abuf, h_out.at[pl.ds(base + row0, _UCH)])
                pltpu.sync_copy(abuf, spbuf.at[pl.ds(row0, _UCH)])

        if gather:
            plsc.subcore_barrier()
            pltpu.sync_copy(src.at[pl.ds(s * _EPT, _EPT)], raw_v)
            _clamped_idx(raw_v, idx_v, base, _ZROW)
            descs2 = [
                pltpu.async_copy(
                    spbuf.at[idx_v.at[j]],
                    dat_v.at[pl.ds(j * _CHK, _CHK)],
                    sem,
                )
                for j in range(_NCHK)
            ]
            for d in descs2:
                d.wait()
            pltpu.sync_copy(dat_v, xj_out.at[c].at[pl.ds(s * _EPT, _EPT)])

    return fused


@functools.cache
def _msgr_fn(ci, co, nparts):
    be = 1024
    grid = _EPAD // be
    bn = 640

    def body(*refs):
        (*xs, eat_ref, w1_ref, b1_ref, w2t_ref, b2_ref,
         h_ref, root_ref, bias_ref, o_ref, r_ref) = refs
        i = pl.program_id(0)
        rows = lax.broadcasted_iota(jnp.int32, (be, 1), 0) + i * be
        valid = rows < _E
        if nparts == 2:
            xj = xs[0][0] + xs[1][0]
        else:
            xj = xs[0][...]
        xj = jnp.where(valid, xj, 0.0)
        validt = (
            lax.broadcasted_iota(jnp.int32, (1, be), 1) + i * be
        ) < _E
        hidt = jnp.maximum(w1_ref[...] * eat_ref[...] + b1_ref[...], 0.0)
        hidt = jnp.where(validt, hidt, 0.0)
        tp = lax.dot_general(
            w2t_ref[...], xj.astype(jnp.bfloat16), (((1,), (1,)), ((), ())),
            preferred_element_type=jnp.float32,
        )
        msgt = hidt[0:1, :] * tp[0:co, :]
        for h in range(1, _HID):
            msgt = msgt + hidt[h:h + 1, :] * tp[h * co:(h + 1) * co, :]
        msg = jnp.transpose(msgt, (1, 0)) + lax.dot_general(
            xj, b2_ref[...], (((1,), (0,)), ((), ())),
            preferred_element_type=jnp.float32,
        )
        o_ref[...] = msg
        r_ref[...] = (
            lax.dot_general(
                h_ref[...], root_ref[...], (((1,), (0,)), ((), ())),
                preferred_element_type=jnp.float32,
                precision=lax.Precision.HIGHEST,
            )
            + bias_ref[...]
        )

    if nparts == 2:
        xspecs = [
            pl.BlockSpec((1, be, ci), lambda i: (0, i, 0)),
            pl.BlockSpec((1, be, ci), lambda i: (1, i, 0)),
        ]
    else:
        xspecs = [pl.BlockSpec((be, ci), lambda i: (i, 0))]
    return pl.pallas_call(
        body,
        grid=(grid,),
        in_specs=xspecs + [
            pl.BlockSpec((1, be), lambda i: (0, i)),
            pl.BlockSpec((_HID, 1), lambda i: (0, 0)),
            pl.BlockSpec((_HID, 1), lambda i: (0, 0)),
            pl.BlockSpec((_HID * co, ci), lambda i: (0, 0)),
            pl.BlockSpec((ci, co), lambda i: (0, 0)),
            pl.BlockSpec((bn, ci), lambda i: (i, 0)),
            pl.BlockSpec((ci, co), lambda i: (0, 0)),
            pl.BlockSpec((1, co), lambda i: (0, 0)),
        ],
        out_specs=[
            pl.BlockSpec((be, co), lambda i: (i, 0)),
            pl.BlockSpec((bn, co), lambda i: (i, 0)),
        ],
        out_shape=[
            jax.ShapeDtypeStruct((_EPAD, co), jnp.float32),
            jax.ShapeDtypeStruct((_N, co), jnp.float32),
        ],
    )


@functools.cache
def _pool_fn(emb, nc):
    def body(h_ref, fcw_ref, fcb_ref, o_ref):
        m = jnp.max(h_ref[...], axis=0, keepdims=True)
        o_ref[...] = (
            lax.dot_general(
                jnp.maximum(m, 0.0), fcw_ref[...], (((1,), (0,)), ((), ())),
                preferred_element_type=jnp.float32,
                precision=lax.Precision.HIGHEST,
            )
            + fcb_ref[...]
        )

    return pl.pallas_call(
        body,
        out_shape=jax.ShapeDtypeStruct((1, nc), jnp.float32),
    )


def kernel(x, edge_attr, params, edge_index, batch):
    pad = _EPAD - _E
    src = jnp.concatenate([edge_index[0], jnp.zeros((pad,), jnp.int32)])
    dst = jnp.concatenate([edge_index[1], jnp.zeros((pad,), jnp.int32)])
    src3d = src.reshape(_NW, _NCHK // 2, _CHK)
    eap = jnp.concatenate([edge_attr, jnp.zeros((pad, 1), jnp.float32)])
    zeros64 = jnp.zeros((_SROWS, 64), jnp.float32)

    h = x
    xparts = [_gather0_fn(_DIMS[0][0])(x, src3d)]
    out = None
    eat = eap.reshape(1, _EPAD)
    for i, (ci, co) in enumerate(_DIMS):
        p = params["l%d" % i]
        w2t = (
            p["w2"].reshape(_HID, ci, co).transpose(0, 2, 1)
            .reshape(_HID * co, ci).astype(jnp.bfloat16)
        )
        msg, r = _msgr_fn(ci, co, len(xparts))(
            *xparts, eat, p["w1"].reshape(_HID, 1), p["b1"].reshape(_HID, 1),
            w2t, p["b2"].reshape(ci, co), h, p["root"],
            p["bias"].reshape(1, co),
        )
        last = i + 1 == len(_DIMS)
        res = _sc_fused_fn(co, not last, not last)(
            msg, dst, src, r, zeros64[:, :co]
        )
        if last:
            h = res[0]
        else:
            h, xj2 = res
            xparts = [xj2, xj2]
    nc = params["fc_w"].shape[1]
    return _pool_fn(64, nc)(h, params["fc_w"], params["fc_b"].reshape(1, nc))


# speedup vs baseline: 3.0808x; 1.0288x over previous
import functools

import jax
import jax.numpy as jnp
from jax import lax
from jax.experimental import pallas as pl
from jax.experimental.pallas import tpu as pltpu
from jax.experimental.pallas import tpu_sc as plsc

_N = 10000
_E = 16000
_HID = 32
_NCORE = 2
_NSUB = 16
_NW = _NCORE * _NSUB
_EPAD = 16384
_EPT = _EPAD // _NSUB
_CHK = 128
_NCHK = _EPT // _CHK
_HALF = _N // _NCORE
_SROWS = 5008
_ZROW = 5000
_TRASH = 5001
_ZCH = _SROWS // _NSUB
_UCH = 125
_NUCH = _HALF // _UCH
_DIMS = ((32, 32), (32, 32), (32, 64), (64, 64), (64, 64))

_mesh = plsc.VectorSubcoreMesh(core_axis_name="c", subcore_axis_name="s")
_sc_params = pltpu.CompilerParams(use_tc_tiling_on_sc=False)


@functools.cache
def _gather0_fn(ci):
    @functools.partial(
        pl.kernel,
        out_type=jax.ShapeDtypeStruct((_EPAD, ci), jnp.float32),
        mesh=_mesh,
        scratch_types=[
            pltpu.VMEM((_NCHK // 2, _CHK), jnp.int32),
            pltpu.VMEM((_EPAD // _NW, ci), jnp.float32),
            pltpu.SemaphoreType.DMA,
        ],
        compiler_params=_sc_params,
    )
    def gat(table, idx, out, idx_v, rows_v, sem):
        epw = _EPAD // _NW
        w = lax.axis_index("s") * _NCORE + lax.axis_index("c")
        pltpu.sync_copy(idx.at[w], idx_v)
        descs = [
            pltpu.async_copy(
                table.at[idx_v.at[j]], rows_v.at[pl.ds(j * _CHK, _CHK)], sem
            )
            for j in range(_NCHK // 2)
        ]
        for d in descs:
            d.wait()
        pltpu.sync_copy(rows_v, out.at[pl.ds(w * epw, epw)])

    return gat


def _clamped_idx(raw_v, idx_v, base, miss_row):
    for k in range(_EPT // 16):
        v = raw_v[pl.ds(k * 16, 16)]
        owned = (v >= base) & (v < base + _HALF)
        loc = jnp.where(owned, v - base, miss_row)
        idx_v[k // 8, pl.ds((k % 8) * 16, 16)] = loc


@functools.cache
def _sc_fused_fn(co, relu, gather):
    out_type = [jax.ShapeDtypeStruct((_N, co), jnp.float32)]
    if gather:
        out_type.append(jax.ShapeDtypeStruct((_NCORE, _EPAD, co), jnp.float32))

    @functools.partial(
        pl.kernel,
        out_type=tuple(out_type),
        mesh=_mesh,
        scratch_types=[
            pltpu.VMEM((_EPT,), jnp.int32),
            pltpu.VMEM((_NCHK, _CHK), jnp.int32),
            pltpu.VMEM((_EPT, co), jnp.float32),
            pltpu.VMEM((_UCH, co), jnp.float32),
            pltpu.VMEM((_UCH, co), jnp.float32),
            pltpu.VMEM_SHARED((_SROWS, co), jnp.float32),
            pltpu.SemaphoreType.DMA,
        ],
        compiler_params=_sc_params,
    )
    def fused(msg, dst, src, r_in, zeros, *rest):
        if gather:
            h_out, xj_out, raw_v, idx_v, dat_v, rbuf, abuf, spbuf, sem = rest
        else:
            h_out, raw_v, idx_v, dat_v, rbuf, abuf, spbuf, sem = rest
        c = lax.axis_index("c")
        s = lax.axis_index("s")
        base = c * _HALF

        d_raw = pltpu.async_copy(dst.at[pl.ds(s * _EPT, _EPT)], raw_v, sem)
        d_msg = pltpu.async_copy(msg.at[pl.ds(s * _EPT, _EPT)], dat_v, sem)
        d_z = pltpu.async_copy(
            zeros.at[pl.ds(s * _ZCH, _ZCH)], spbuf.at[pl.ds(s * _ZCH, _ZCH)],
            sem,
        )
        d_raw.wait()
        _clamped_idx(raw_v, idx_v, base, _TRASH)
        d_msg.wait()
        d_z.wait()
        plsc.subcore_barrier()

        descs = [
            pltpu.async_copy(
                dat_v.at[pl.ds(j * _CHK, _CHK)],
                spbuf.at[idx_v.at[j]],
                sem,
                add=True,
            )
            for j in range(_NCHK)
        ]
        for d in descs:
            d.wait()
        plsc.subcore_barrier()

        for k in range(_NUCH // _NSUB + 1):
            cid = s + _NSUB * k

            @pl.when(cid < _NUCH)
            def _():
                row0 = cid * _UCH
                pltpu.sync_copy(r_in.at[pl.ds(base + row0, _UCH)], rbuf)
                pltpu.sync_copy(spbuf.at[pl.ds(row0, _UCH)], abuf)

                def upd(i, carry):
                    for l in range(co // 16):
                        vv = (
                            abuf[i, pl.ds(l * 16, 16)]
                            + rbuf[i, pl.ds(l * 16, 16)]
                        )
                        if relu:
                            vv = jnp.maximum(vv, 0.0)
                        abuf[i, pl.ds(l * 16, 16)] = vv
                    return carry

                lax.fori_loop(0, _UCH, upd, 0)
                pltpu.sync_copy(abuf, h_out.at[pl.ds(base + row0, _UCH)])
                pltpu.sync_copy(abuf, spbuf.at[pl.ds(row0, _UCH)])

        if gather:
            plsc.subcore_barrier()
            pltpu.sync_copy(src.at[pl.ds(s * _EPT, _EPT)], raw_v)
            _clamped_idx(raw_v, idx_v, base, _ZROW)
            descs2 = [
                pltpu.async_copy(
                    spbuf.at[idx_v.at[j]],
                    dat_v.at[pl.ds(j * _CHK, _CHK)],
                    sem,
                )
                for j in range(_NCHK)
            ]
            for d in descs2:
                d.wait()
            pltpu.sync_copy(dat_v, xj_out.at[c].at[pl.ds(s * _EPT, _EPT)])

    return fused


@functools.cache
def _msgr_fn(ci, co, nparts):
    be = 1024
    grid = _EPAD // be
    bn = 640

    def body(*refs):
        (*xs, eat_ref, w1_ref, b1_ref, w2t_ref, b2_ref,
         h_ref, root_ref, bias_ref, o_ref, r_ref) = refs
        i = pl.program_id(0)
        rows = lax.broadcasted_iota(jnp.int32, (be, 1), 0) + i * be
        valid = rows < _E
        if nparts == 2:
            xj = xs[0][0] + xs[1][0]
        else:
            xj = xs[0][...]
        xj = jnp.where(valid, xj, 0.0)
        validt = (
            lax.broadcasted_iota(jnp.int32, (1, be), 1) + i * be
        ) < _E
        hidt = jnp.maximum(w1_ref[...] * eat_ref[...] + b1_ref[...], 0.0)
        hidt = jnp.where(validt, hidt, 0.0)
        tp = lax.dot_general(
            w2t_ref[...], xj.astype(jnp.bfloat16), (((1,), (1,)), ((), ())),
            preferred_element_type=jnp.float32,
        )
        msgt = hidt[0:1, :] * tp[0:co, :]
        for h in range(1, _HID):
            msgt = msgt + hidt[h:h + 1, :] * tp[h * co:(h + 1) * co, :]
        msg = jnp.transpose(msgt, (1, 0)) + lax.dot_general(
            xj, b2_ref[...], (((1,), (0,)), ((), ())),
            preferred_element_type=jnp.float32,
        )
        o_ref[...] = msg
        r_ref[...] = (
            lax.dot_general(
                h_ref[...], root_ref[...], (((1,), (0,)), ((), ())),
                preferred_element_type=jnp.float32,
            )
            + bias_ref[...]
        )

    if nparts == 2:
        xspecs = [
            pl.BlockSpec((1, be, ci), lambda i: (0, i, 0)),
            pl.BlockSpec((1, be, ci), lambda i: (1, i, 0)),
        ]
    else:
        xspecs = [pl.BlockSpec((be, ci), lambda i: (i, 0))]
    return pl.pallas_call(
        body,
        grid=(grid,),
        in_specs=xspecs + [
            pl.BlockSpec((1, be), lambda i: (0, i)),
            pl.BlockSpec((_HID, 1), lambda i: (0, 0)),
            pl.BlockSpec((_HID, 1), lambda i: (0, 0)),
            pl.BlockSpec((_HID * co, ci), lambda i: (0, 0)),
            pl.BlockSpec((ci, co), lambda i: (0, 0)),
            pl.BlockSpec((bn, ci), lambda i: (i, 0)),
            pl.BlockSpec((ci, co), lambda i: (0, 0)),
            pl.BlockSpec((1, co), lambda i: (0, 0)),
        ],
        out_specs=[
            pl.BlockSpec((be, co), lambda i: (i, 0)),
            pl.BlockSpec((bn, co), lambda i: (i, 0)),
        ],
        out_shape=[
            jax.ShapeDtypeStruct((_EPAD, co), jnp.float32),
            jax.ShapeDtypeStruct((_N, co), jnp.float32),
        ],
    )


@functools.cache
def _pool_fn(emb, nc):
    def body(h_ref, fcw_ref, fcb_ref, o_ref):
        m = jnp.max(h_ref[...], axis=0, keepdims=True)
        o_ref[...] = (
            lax.dot_general(
                jnp.maximum(m, 0.0), fcw_ref[...], (((1,), (0,)), ((), ())),
                preferred_element_type=jnp.float32,
                precision=lax.Precision.HIGHEST,
            )
            + fcb_ref[...]
        )

    return pl.pallas_call(
        body,
        out_shape=jax.ShapeDtypeStruct((1, nc), jnp.float32),
    )


def kernel(x, edge_attr, params, edge_index, batch):
    pad = _EPAD - _E
    src = jnp.concatenate([edge_index[0], jnp.zeros((pad,), jnp.int32)])
    dst = jnp.concatenate([edge_index[1], jnp.zeros((pad,), jnp.int32)])
    src3d = src.reshape(_NW, _NCHK // 2, _CHK)
    eap = jnp.concatenate([edge_attr, jnp.zeros((pad, 1), jnp.float32)])
    zeros64 = jnp.zeros((_SROWS, 64), jnp.float32)

    h = x
    xparts = [_gather0_fn(_DIMS[0][0])(x, src3d)]
    out = None
    eat = eap.reshape(1, _EPAD)
    for i, (ci, co) in enumerate(_DIMS):
        p = params["l%d" % i]
        w2t = (
            p["w2"].reshape(_HID, ci, co).transpose(0, 2, 1)
            .reshape(_HID * co, ci).astype(jnp.bfloat16)
        )
        msg, r = _msgr_fn(ci, co, len(xparts))(
            *xparts, eat, p["w1"].reshape(_HID, 1), p["b1"].reshape(_HID, 1),
            w2t, p["b2"].reshape(ci, co), h, p["root"],
            p["bias"].reshape(1, co),
        )
        last = i + 1 == len(_DIMS)
        res = _sc_fused_fn(co, not last, not last)(
            msg, dst, src, r, zeros64[:, :co]
        )
        if last:
            h = res[0]
        else:
            h, xj2 = res
            xparts = [xj2, xj2]
    nc = params["fc_w"].shape[1]
    return _pool_fn(64, nc)(h, params["fc_w"], params["fc_b"].reshape(1, nc))
